# baseline (speedup 1.0000x reference)
import functools
import jax
import jax.numpy as jnp
from jax import lax
from jax.experimental import pallas as pl
from jax.experimental.pallas import tpu as pltpu
from jax.experimental.pallas import tpu_sc as plsc

_E = 8
_T = 256
_GTM = 512
_CTM = 512
_NW = 32
_SUB = 16
_NGID = 48


def _gating_body(x_ref, wg_ref, xf_ref, w1_ref, w2_ref, i1_ref, i2_ref,
                 r1_ref, r2_ref, cnt_ref, cacc_ref):
    pid = pl.program_id(0)

    @pl.when(pid == 0)
    def _init():
        cacc_ref[...] = jnp.zeros_like(cacc_ref)

    xt = x_ref[0].T
    xf_ref[...] = xt
    logits = lax.dot_general(xt, wg_ref[...], (((1,), (1,)), ((), ())),
                             preferred_element_type=jnp.float32)
    m = jnp.max(logits, axis=1, keepdims=True)
    ex = jnp.exp(logits - m)
    p = ex / jnp.sum(ex, axis=1, keepdims=True)
    lanes = lax.broadcasted_iota(jnp.int32, p.shape, 1)
    m1 = jnp.max(p, axis=1, keepdims=True)
    i1 = jnp.min(jnp.where(p == m1, lanes, _E), axis=1, keepdims=True)
    p2 = jnp.where(lanes == i1, -jnp.inf, p)
    m2 = jnp.max(p2, axis=1, keepdims=True)
    i2 = jnp.min(jnp.where(p2 == m2, lanes, _E), axis=1, keepdims=True)

    oh1 = (lanes == i1).astype(jnp.float32)
    oh2 = (lanes == i2).astype(jnp.float32)
    ohc = oh1 + oh2
    rows = lax.broadcasted_iota(jnp.int32, (_GTM, _GTM), 0)
    cols = lax.broadcasted_iota(jnp.int32, (_GTM, _GTM), 1)
    tril = (cols < rows).astype(jnp.float32)
    ranks = lax.dot_general(tril, ohc, (((1,), (0,)), ((), ())),
                            preferred_element_type=jnp.float32)
    ranks = ranks + cacc_ref[...]
    r1 = jnp.sum(ranks * oh1, axis=1, keepdims=True)
    r2 = jnp.sum(ranks * oh2, axis=1, keepdims=True)

    w1_ref[...] = m1
    w2_ref[...] = m2
    i1_ref[...] = i1
    i2_ref[...] = i2
    r1_ref[...] = r1.astype(jnp.int32)
    r2_ref[...] = r2.astype(jnp.int32)
    cacc_ref[...] += jnp.sum(ohc, axis=0, keepdims=True)
    cnt_ref[...] = cacc_ref[...]


def _gating(x, Wg):
    B, in_dim, n_vars = x.shape
    n_tok = B * n_vars
    vpb = n_vars // _GTM
    grid = (n_tok // _GTM,)
    f32 = jnp.float32
    i32 = jnp.int32
    outs = [jax.ShapeDtypeStruct((n_tok, in_dim), f32),
            jax.ShapeDtypeStruct((n_tok, 1), f32),
            jax.ShapeDtypeStruct((n_tok, 1), f32),
            jax.ShapeDtypeStruct((n_tok, 1), i32),
            jax.ShapeDtypeStruct((n_tok, 1), i32),
            jax.ShapeDtypeStruct((n_tok, 1), i32),
            jax.ShapeDtypeStruct((n_tok, 1), i32),
            jax.ShapeDtypeStruct((1, _E), f32)]
    tok_spec = pl.BlockSpec((_GTM, 1), lambda i: (i, 0))
    return pl.pallas_call(
        _gating_body,
        grid=grid,
        in_specs=[pl.BlockSpec((1, in_dim, _GTM),
                               lambda i: (i // vpb, 0, i % vpb)),
                  pl.BlockSpec((_E, in_dim), lambda i: (0, 0))],
        out_specs=[pl.BlockSpec((_GTM, in_dim), lambda i: (i, 0)),
                   tok_spec, tok_spec, tok_spec, tok_spec, tok_spec, tok_spec,
                   pl.BlockSpec((1, _E), lambda i: (0, 0))],
        out_shape=outs,
        scratch_shapes=[pltpu.VMEM((1, _E), f32)],
    )(x, Wg)


_TSHIFT = _T.bit_length() - 1


def _aligned_offsets(cv_ref):
    ci = cv_ref[...].astype(jnp.int32)
    av = ((ci + (_T - 1)) >> _TSHIFT) << _TSHIFT
    offs, acc = [], 0
    for e in range(_E):
        offs.append(acc)
        acc = acc + av[e]
    return offs, acc


def _slot_chunk(offs, iv_ref, rv_ref, j):
    ic = iv_ref[pl.ds(j * _SUB, _SUB)]
    s = rv_ref[pl.ds(j * _SUB, _SUB)]
    for e in range(_E):
        s = s + jnp.where(ic == e, offs[e], 0)
    return s


def _sc_scatter_x(xf, i1, i2, r1, r2, c16, n_slots):
    n_tok, d = xf.shape
    per_w = n_tok // _NW
    nsub = per_w // _SUB
    mesh = plsc.VectorSubcoreMesh(core_axis_name="c", subcore_axis_name="s")
    i32 = jnp.int32
    f32 = jnp.float32

    @functools.partial(
        pl.kernel, mesh=mesh,
        out_type=[jax.ShapeDtypeStruct((n_slots, d), f32),
                  jax.ShapeDtypeStruct((_NGID,), i32)],
        scratch_types=[pltpu.VMEM((per_w,), i32), pltpu.VMEM((per_w,), i32),
                       pltpu.VMEM((per_w,), i32), pltpu.VMEM((per_w,), i32),
                       pltpu.VMEM((16,), f32), pltpu.VMEM((_NGID,), i32),
                       pltpu.VMEM((_SUB, d), f32), pltpu.VMEM((_SUB, d), f32),
                       pltpu.SemaphoreType.DMA, pltpu.SemaphoreType.DMA,
                       pltpu.SemaphoreType.DMA],
    )
    def k(x_hbm, i1_hbm, i2_hbm, r1_hbm, r2_hbm, c16_hbm, xs_hbm, gid_hbm,
          i1v, i2v, r1v, r2v, cv, gidv, xv0, xv1,
          lsem, ssem0, ssem1):
        wid = lax.axis_index("s") * 2 + lax.axis_index("c")
        base = wid * per_w
        pltpu.sync_copy(c16_hbm, cv)
        offs, total = _aligned_offsets(cv)
        pltpu.sync_copy(i1_hbm.at[pl.ds(base, per_w)], i1v)
        pltpu.sync_copy(i2_hbm.at[pl.ds(base, per_w)], i2v)
        pltpu.sync_copy(r1_hbm.at[pl.ds(base, per_w)], r1v)
        pltpu.sync_copy(r2_hbm.at[pl.ds(base, per_w)], r2v)

        @pl.when(wid == 0)
        def _gids():
            for c in range(_NGID // 16):
                tstart = (lax.broadcasted_iota(i32, (16,), 0) + c * 16) * _T
                g = jnp.full((16,), -1, i32)
                for e in range(_E):
                    g = g + jnp.where(tstart >= offs[e], 1, 0)
                gidv[pl.ds(c * 16, 16)] = g
            pltpu.sync_copy(gidv, gid_hbm)

        xbufs = (xv0, xv1)
        ssems = (ssem0, ssem1)
        loads = [pltpu.async_copy(x_hbm.at[pl.ds(base, _SUB)], xv0, lsem)]
        scats = []
        for j in range(nsub):
            loads[j].wait()
            s1c = _slot_chunk(offs, i1v, r1v, j)
            s2c = _slot_chunk(offs, i2v, r2v, j)
            xb = xbufs[j % 2]
            scats.append((
                pltpu.async_copy(xb, xs_hbm.at[s1c], ssems[j % 2]),
                pltpu.async_copy(xb, xs_hbm.at[s2c], ssems[j % 2])))
            if j + 1 < nsub:
                if j >= 1:
                    scats[j - 1][0].wait()
                    scats[j - 1][1].wait()
                loads.append(pltpu.async_copy(
                    x_hbm.at[pl.ds(base + (j + 1) * _SUB, _SUB)],
                    xbufs[(j + 1) % 2], lsem))
        for jj in (nsub - 2, nsub - 1):
            if 0 <= jj < nsub:
                scats[jj][0].wait()
                scats[jj][1].wait()

    return k(xf, i1, i2, r1, r2, c16)


def _sc_gather_buf(buf, i12, r12, c16):
    n_slots, d = buf.shape
    n_asn = i12.shape[0]
    per_w = n_asn // _NW
    nsub = per_w // _SUB
    mesh = plsc.VectorSubcoreMesh(core_axis_name="c", subcore_axis_name="s")
    i32 = jnp.int32
    f32 = jnp.float32

    @functools.partial(
        pl.kernel, mesh=mesh,
        out_type=jax.ShapeDtypeStruct((n_asn, d), f32),
        scratch_types=[pltpu.VMEM((per_w,), i32), pltpu.VMEM((per_w,), i32),
                       pltpu.VMEM((16,), f32),
                       pltpu.VMEM((_SUB, d), f32), pltpu.VMEM((_SUB, d), f32),
                       pltpu.SemaphoreType.DMA, pltpu.SemaphoreType.DMA,
                       pltpu.SemaphoreType.DMA, pltpu.SemaphoreType.DMA],
    )
    def k(buf_hbm, i12_hbm, r12_hbm, c16_hbm, g12_hbm,
          i12v, r12v, cv, gv0, gv1, gsem0, gsem1, stsem0, stsem1):
        wid = lax.axis_index("s") * 2 + lax.axis_index("c")
        base = wid * per_w
        pltpu.sync_copy(c16_hbm, cv)
        offs, _ = _aligned_offsets(cv)
        pltpu.sync_copy(i12_hbm.at[pl.ds(base, per_w)], i12v)
        pltpu.sync_copy(r12_hbm.at[pl.ds(base, per_w)], r12v)

        gbufs = (gv0, gv1)
        gsems = (gsem0, gsem1)
        stsems = (stsem0, stsem1)
        s0 = _slot_chunk(offs, i12v, r12v, 0)
        gaths = [pltpu.async_copy(buf_hbm.at[s0], gv0, gsem0)]
        stores = []
        for j in range(nsub):
            gaths[j].wait()
            if j + 1 < nsub:
                if j >= 1:
                    stores[j - 1].wait()
                sc = _slot_chunk(offs, i12v, r12v, j + 1)
                gaths.append(pltpu.async_copy(
                    buf_hbm.at[sc], gbufs[(j + 1) % 2], gsems[(j + 1) % 2]))
            stores.append(pltpu.async_copy(
                gbufs[j % 2], g12_hbm.at[pl.ds(base + j * _SUB, _SUB)],
                stsems[j % 2]))
        for jj in (nsub - 2, nsub - 1):
            if 0 <= jj < nsub:
                stores[jj].wait()

    return k(buf, i12, r12, c16)


def _gmm_body(gid_ref, xs_ref, we_ref, be_ref, out_ref):
    acc = lax.dot_general(xs_ref[...], we_ref[0],
                          (((1,), (1,)), ((), ())),
                          preferred_element_type=jnp.float32)
    out_ref[...] = acc + be_ref[0]


def _grouped_matmul(gids, Xs, We, be3):
    n_slots, in_dim = Xs.shape
    out_dim = We.shape[1]
    n_tiles = n_slots // _T
    we_map = lambda i, g: (g[i], 0, 0)
    gspec = pltpu.PrefetchScalarGridSpec(
        num_scalar_prefetch=1,
        grid=(n_tiles,),
        in_specs=[
            pl.BlockSpec((_T, in_dim), lambda i, g: (i, 0)),
            pl.BlockSpec((1, out_dim, in_dim), we_map),
            pl.BlockSpec((1, 1, out_dim), we_map),
        ],
        out_specs=pl.BlockSpec((_T, out_dim), lambda i, g: (i, 0)),
    )
    return pl.pallas_call(
        _gmm_body,
        grid_spec=gspec,
        out_shape=jax.ShapeDtypeStruct((n_slots, out_dim), jnp.float32),
    )(gids, Xs, We, be3)


def _combine_body(g12_ref, w1_ref, w2_ref, out_ref):
    y = w1_ref[...] * g12_ref[:, 0, :] + w2_ref[...] * g12_ref[:, 1, :]
    out_ref[0] = y.T


def _combine(g12, w1, w2, B, n_vars):
    n_tok, _, d = g12.shape
    tm = _CTM
    vpb = n_vars // tm
    tok_spec = pl.BlockSpec((tm, 1), lambda i: (i, 0))
    return pl.pallas_call(
        _combine_body,
        grid=(n_tok // tm,),
        in_specs=[pl.BlockSpec((tm, 2, d), lambda i: (i, 0, 0)),
                  tok_spec, tok_spec],
        out_specs=pl.BlockSpec((1, d, tm), lambda i: (i // vpb, 0, i % vpb)),
        out_shape=jax.ShapeDtypeStruct((B, d, n_vars), jnp.float32),
    )(g12, w1, w2)


def kernel(x, Wg, We, be):
    B, in_len, n_vars = x.shape
    out_dim = We.shape[1]
    n_tok = B * n_vars
    n_slots = 2 * n_tok + _E * _T

    xf, w1, w2, i1, i2, r1, r2, cnt = _gating(x, Wg)
    c16 = jnp.pad(cnt.reshape(_E), (0, 16 - _E))
    Xs, gids = _sc_scatter_x(xf, i1.reshape(n_tok), i2.reshape(n_tok),
                             r1.reshape(n_tok), r2.reshape(n_tok),
                             c16, n_slots)
    be3 = be.reshape(_E, 1, out_dim)
    buf = _grouped_matmul(gids, Xs, We, be3)
    i12 = jnp.concatenate([i1, i2], axis=1).reshape(2 * n_tok)
    r12 = jnp.concatenate([r1, r2], axis=1).reshape(2 * n_tok)
    g12f = _sc_gather_buf(buf, i12, r12, c16)
    g12 = g12f.reshape(n_tok, 2, out_dim)
    return _combine(g12, w1, w2, B, n_vars)


# speedup vs baseline: 1.2479x; 1.2479x over previous
import functools
import jax
import jax.numpy as jnp
from jax import lax
from jax.experimental import pallas as pl
from jax.experimental.pallas import tpu as pltpu
from jax.experimental.pallas import tpu_sc as plsc

_E = 8
_T = 256
_GTM = 512
_CTM = 512
_NW = 32
_SUB = 16
_NGID = 48


def _gating_body(x_ref, wg_ref, xf_ref, w1_ref, w2_ref, i1_ref, i2_ref,
                 r1_ref, r2_ref, cnt_ref, cacc_ref):
    pid = pl.program_id(0)

    @pl.when(pid == 0)
    def _init():
        cacc_ref[...] = jnp.zeros_like(cacc_ref)

    xt = x_ref[0].T
    xf_ref[...] = xt
    logits = lax.dot_general(xt, wg_ref[...], (((1,), (1,)), ((), ())),
                             preferred_element_type=jnp.float32)
    m = jnp.max(logits, axis=1, keepdims=True)
    ex = jnp.exp(logits - m)
    p = ex / jnp.sum(ex, axis=1, keepdims=True)
    lanes = lax.broadcasted_iota(jnp.int32, p.shape, 1)
    m1 = jnp.max(p, axis=1, keepdims=True)
    i1 = jnp.min(jnp.where(p == m1, lanes, _E), axis=1, keepdims=True)
    p2 = jnp.where(lanes == i1, -jnp.inf, p)
    m2 = jnp.max(p2, axis=1, keepdims=True)
    i2 = jnp.min(jnp.where(p2 == m2, lanes, _E), axis=1, keepdims=True)

    oh1 = (lanes == i1).astype(jnp.float32)
    oh2 = (lanes == i2).astype(jnp.float32)
    ohc = oh1 + oh2
    rows = lax.broadcasted_iota(jnp.int32, (_GTM, _GTM), 0)
    cols = lax.broadcasted_iota(jnp.int32, (_GTM, _GTM), 1)
    tril = (cols < rows).astype(jnp.float32)
    ranks = lax.dot_general(tril, ohc, (((1,), (0,)), ((), ())),
                            preferred_element_type=jnp.float32)
    ranks = ranks + cacc_ref[...]
    r1 = jnp.sum(ranks * oh1, axis=1, keepdims=True)
    r2 = jnp.sum(ranks * oh2, axis=1, keepdims=True)

    w1_ref[...] = m1
    w2_ref[...] = m2
    i1_ref[...] = i1
    i2_ref[...] = i2
    r1_ref[...] = r1.astype(jnp.int32)
    r2_ref[...] = r2.astype(jnp.int32)
    cacc_ref[...] += jnp.sum(ohc, axis=0, keepdims=True)
    cnt_ref[...] = cacc_ref[...]


def _gating(x, Wg):
    B, in_dim, n_vars = x.shape
    n_tok = B * n_vars
    vpb = n_vars // _GTM
    grid = (n_tok // _GTM,)
    f32 = jnp.float32
    i32 = jnp.int32
    outs = [jax.ShapeDtypeStruct((n_tok, in_dim), f32),
            jax.ShapeDtypeStruct((n_tok, 1), f32),
            jax.ShapeDtypeStruct((n_tok, 1), f32),
            jax.ShapeDtypeStruct((n_tok, 1), i32),
            jax.ShapeDtypeStruct((n_tok, 1), i32),
            jax.ShapeDtypeStruct((n_tok, 1), i32),
            jax.ShapeDtypeStruct((n_tok, 1), i32),
            jax.ShapeDtypeStruct((1, _E), f32)]
    tok_spec = pl.BlockSpec((_GTM, 1), lambda i: (i, 0))
    return pl.pallas_call(
        _gating_body,
        grid=grid,
        in_specs=[pl.BlockSpec((1, in_dim, _GTM),
                               lambda i: (i // vpb, 0, i % vpb)),
                  pl.BlockSpec((_E, in_dim), lambda i: (0, 0))],
        out_specs=[pl.BlockSpec((_GTM, in_dim), lambda i: (i, 0)),
                   tok_spec, tok_spec, tok_spec, tok_spec, tok_spec, tok_spec,
                   pl.BlockSpec((1, _E), lambda i: (0, 0))],
        out_shape=outs,
        scratch_shapes=[pltpu.VMEM((1, _E), f32)],
    )(x, Wg)


_TSHIFT = _T.bit_length() - 1


def _aligned_offsets(cv_ref):
    ci = cv_ref[...].astype(jnp.int32)
    av = ((ci + (_T - 1)) >> _TSHIFT) << _TSHIFT
    offs, acc = [], 0
    for e in range(_E):
        offs.append(acc)
        acc = acc + av[e]
    return offs, acc


def _slot_chunk(offs, iv_ref, rv_ref, j):
    ic = iv_ref[pl.ds(j * _SUB, _SUB)]
    s = rv_ref[pl.ds(j * _SUB, _SUB)]
    for e in range(_E):
        s = s + jnp.where(ic == e, offs[e], 0)
    return s


def _sc_scatter_x(xf, i1, i2, r1, r2, c16, n_slots):
    n_tok, d = xf.shape
    per_w = n_tok // _NW
    nsub = per_w // _SUB
    mesh = plsc.VectorSubcoreMesh(core_axis_name="c", subcore_axis_name="s")
    i32 = jnp.int32
    f32 = jnp.float32

    @functools.partial(
        pl.kernel, mesh=mesh,
        out_type=[jax.ShapeDtypeStruct((n_slots, d), f32),
                  jax.ShapeDtypeStruct((_NGID,), i32)],
        scratch_types=[pltpu.VMEM((per_w,), i32), pltpu.VMEM((per_w,), i32),
                       pltpu.VMEM((per_w,), i32), pltpu.VMEM((per_w,), i32),
                       pltpu.VMEM((16,), f32), pltpu.VMEM((_NGID,), i32),
                       pltpu.VMEM((_SUB, d), f32), pltpu.VMEM((_SUB, d), f32),
                       pltpu.SemaphoreType.DMA, pltpu.SemaphoreType.DMA,
                       pltpu.SemaphoreType.DMA],
    )
    def k(x_hbm, i1_hbm, i2_hbm, r1_hbm, r2_hbm, c16_hbm, xs_hbm, gid_hbm,
          i1v, i2v, r1v, r2v, cv, gidv, xv0, xv1,
          lsem, ssem0, ssem1):
        wid = lax.axis_index("s") * 2 + lax.axis_index("c")
        base = wid * per_w
        pltpu.sync_copy(c16_hbm, cv)
        offs, total = _aligned_offsets(cv)
        pltpu.sync_copy(i1_hbm.at[pl.ds(base, per_w)], i1v)
        pltpu.sync_copy(i2_hbm.at[pl.ds(base, per_w)], i2v)
        pltpu.sync_copy(r1_hbm.at[pl.ds(base, per_w)], r1v)
        pltpu.sync_copy(r2_hbm.at[pl.ds(base, per_w)], r2v)

        @pl.when(wid == 0)
        def _gids():
            for c in range(_NGID // 16):
                tstart = (lax.broadcasted_iota(i32, (16,), 0) + c * 16) * _T
                g = jnp.full((16,), -1, i32)
                for e in range(_E):
                    g = g + jnp.where(tstart >= offs[e], 1, 0)
                gidv[pl.ds(c * 16, 16)] = g
            pltpu.sync_copy(gidv, gid_hbm)

        xbufs = (xv0, xv1)
        ssems = (ssem0, ssem1)
        loads = [pltpu.async_copy(x_hbm.at[pl.ds(base, _SUB)], xv0, lsem)]
        scats = []
        for j in range(nsub):
            loads[j].wait()
            s1c = _slot_chunk(offs, i1v, r1v, j)
            s2c = _slot_chunk(offs, i2v, r2v, j)
            xb = xbufs[j % 2]
            scats.append((
                pltpu.async_copy(xb, xs_hbm.at[s1c], ssems[j % 2]),
                pltpu.async_copy(xb, xs_hbm.at[s2c], ssems[j % 2])))
            if j + 1 < nsub:
                if j >= 1:
                    scats[j - 1][0].wait()
                    scats[j - 1][1].wait()
                loads.append(pltpu.async_copy(
                    x_hbm.at[pl.ds(base + (j + 1) * _SUB, _SUB)],
                    xbufs[(j + 1) % 2], lsem))
        for jj in (nsub - 2, nsub - 1):
            if 0 <= jj < nsub:
                scats[jj][0].wait()
                scats[jj][1].wait()

    return k(xf, i1, i2, r1, r2, c16)


def _sc_gather_buf(buf, i1, i2, r1, r2, c16):
    n_slots, d = buf.shape
    n_tok = i1.shape[0]
    per_w = n_tok // _NW
    nsub = 2 * (per_w // _SUB)
    mesh = plsc.VectorSubcoreMesh(core_axis_name="c", subcore_axis_name="s")
    i32 = jnp.int32
    f32 = jnp.float32

    @functools.partial(
        pl.kernel, mesh=mesh,
        out_type=jax.ShapeDtypeStruct((2 * n_tok, d), f32),
        scratch_types=[pltpu.VMEM((per_w,), i32), pltpu.VMEM((per_w,), i32),
                       pltpu.VMEM((per_w,), i32), pltpu.VMEM((per_w,), i32),
                       pltpu.VMEM((16,), f32),
                       pltpu.VMEM((_SUB, d), f32), pltpu.VMEM((_SUB, d), f32),
                       pltpu.SemaphoreType.DMA, pltpu.SemaphoreType.DMA,
                       pltpu.SemaphoreType.DMA, pltpu.SemaphoreType.DMA],
    )
    def k(buf_hbm, i1_hbm, i2_hbm, r1_hbm, r2_hbm, c16_hbm, g12_hbm,
          i1v, i2v, r1v, r2v, cv, gv0, gv1, gsem0, gsem1, stsem0, stsem1):
        wid = lax.axis_index("s") * 2 + lax.axis_index("c")
        base = wid * per_w
        pltpu.sync_copy(c16_hbm, cv)
        offs, _ = _aligned_offsets(cv)
        pltpu.sync_copy(i1_hbm.at[pl.ds(base, per_w)], i1v)
        pltpu.sync_copy(i2_hbm.at[pl.ds(base, per_w)], i2v)
        pltpu.sync_copy(r1_hbm.at[pl.ds(base, per_w)], r1v)
        pltpu.sync_copy(r2_hbm.at[pl.ds(base, per_w)], r2v)

        def slot(j):
            if j % 2 == 0:
                return _slot_chunk(offs, i1v, r1v, j // 2)
            return _slot_chunk(offs, i2v, r2v, j // 2)

        def dst(j):
            half = 0 if j % 2 == 0 else n_tok
            return pl.ds(half + base + (j // 2) * _SUB, _SUB)

        gbufs = (gv0, gv1)
        gsems = (gsem0, gsem1)
        stsems = (stsem0, stsem1)
        gaths = [pltpu.async_copy(buf_hbm.at[slot(0)], gv0, gsem0)]
        stores = []
        for j in range(nsub):
            gaths[j].wait()
            if j + 1 < nsub:
                if j >= 1:
                    stores[j - 1].wait()
                gaths.append(pltpu.async_copy(
                    buf_hbm.at[slot(j + 1)], gbufs[(j + 1) % 2],
                    gsems[(j + 1) % 2]))
            stores.append(pltpu.async_copy(
                gbufs[j % 2], g12_hbm.at[dst(j)], stsems[j % 2]))
        for jj in (nsub - 2, nsub - 1):
            if 0 <= jj < nsub:
                stores[jj].wait()

    return k(buf, i1, i2, r1, r2, c16)


def _gmm_body(gid_ref, xs_ref, we_ref, be_ref, out_ref):
    acc = lax.dot_general(xs_ref[...], we_ref[0],
                          (((1,), (1,)), ((), ())),
                          preferred_element_type=jnp.float32)
    out_ref[...] = acc + be_ref[0]


def _grouped_matmul(gids, Xs, We, be3):
    n_slots, in_dim = Xs.shape
    out_dim = We.shape[1]
    n_tiles = n_slots // _T
    we_map = lambda i, g: (g[i], 0, 0)
    gspec = pltpu.PrefetchScalarGridSpec(
        num_scalar_prefetch=1,
        grid=(n_tiles,),
        in_specs=[
            pl.BlockSpec((_T, in_dim), lambda i, g: (i, 0)),
            pl.BlockSpec((1, out_dim, in_dim), we_map),
            pl.BlockSpec((1, 1, out_dim), we_map),
        ],
        out_specs=pl.BlockSpec((_T, out_dim), lambda i, g: (i, 0)),
    )
    return pl.pallas_call(
        _gmm_body,
        grid_spec=gspec,
        out_shape=jax.ShapeDtypeStruct((n_slots, out_dim), jnp.float32),
    )(gids, Xs, We, be3)


def _combine_body(g1_ref, g2_ref, w1_ref, w2_ref, out_ref):
    y = w1_ref[...] * g1_ref[...] + w2_ref[...] * g2_ref[...]
    out_ref[0] = y.T


def _combine(g12, w1, w2, B, n_vars):
    n2, d = g12.shape
    n_tok = n2 // 2
    tm = _CTM
    vpb = n_vars // tm
    nb = n_tok // tm
    tok_spec = pl.BlockSpec((tm, 1), lambda i: (i, 0))
    return pl.pallas_call(
        _combine_body,
        grid=(nb,),
        in_specs=[pl.BlockSpec((tm, d), lambda i: (i, 0)),
                  pl.BlockSpec((tm, d), lambda i: (i + nb, 0)),
                  tok_spec, tok_spec],
        out_specs=pl.BlockSpec((1, d, tm), lambda i: (i // vpb, 0, i % vpb)),
        out_shape=jax.ShapeDtypeStruct((B, d, n_vars), jnp.float32),
    )(g12, g12, w1, w2)


def kernel(x, Wg, We, be):
    B, in_len, n_vars = x.shape
    out_dim = We.shape[1]
    n_tok = B * n_vars
    n_slots = 2 * n_tok + _E * _T

    xf, w1, w2, i1, i2, r1, r2, cnt = _gating(x, Wg)
    c16 = jnp.pad(cnt.reshape(_E), (0, 16 - _E))
    Xs, gids = _sc_scatter_x(xf, i1.reshape(n_tok), i2.reshape(n_tok),
                             r1.reshape(n_tok), r2.reshape(n_tok),
                             c16, n_slots)
    be3 = be.reshape(_E, 1, out_dim)
    buf = _grouped_matmul(gids, Xs, We, be3)
    g12 = _sc_gather_buf(buf, i1.reshape(n_tok), i2.reshape(n_tok),
                         r1.reshape(n_tok), r2.reshape(n_tok), c16)
    return _combine(g12, w1, w2, B, n_vars)


# speedup vs baseline: 1.2862x; 1.0307x over previous
import functools
import jax
import jax.numpy as jnp
from jax import lax
from jax.experimental import pallas as pl
from jax.experimental.pallas import tpu as pltpu
from jax.experimental.pallas import tpu_sc as plsc

_E = 8
_T = 256
_GTM = 512
_CTM = 512
_NW = 32
_SUB = 16
_NGID = 48


def _gating_body(x_ref, wg_ref, xf_ref, w1_ref, w2_ref, i1_ref, i2_ref,
                 r1_ref, r2_ref, cnt_ref, cacc_ref):
    pid = pl.program_id(0)

    @pl.when(pid == 0)
    def _init():
        cacc_ref[...] = jnp.zeros_like(cacc_ref)

    xt = x_ref[0].T
    xf_ref[...] = xt
    logits = lax.dot_general(xt, wg_ref[...], (((1,), (1,)), ((), ())),
                             preferred_element_type=jnp.float32)
    m = jnp.max(logits, axis=1, keepdims=True)
    ex = jnp.exp(logits - m)
    p = ex / jnp.sum(ex, axis=1, keepdims=True)
    lanes = lax.broadcasted_iota(jnp.int32, p.shape, 1)
    m1 = jnp.max(p, axis=1, keepdims=True)
    i1 = jnp.min(jnp.where(p == m1, lanes, _E), axis=1, keepdims=True)
    p2 = jnp.where(lanes == i1, -jnp.inf, p)
    m2 = jnp.max(p2, axis=1, keepdims=True)
    i2 = jnp.min(jnp.where(p2 == m2, lanes, _E), axis=1, keepdims=True)

    oh1 = (lanes == i1).astype(jnp.float32)
    oh2 = (lanes == i2).astype(jnp.float32)
    ohc = oh1 + oh2
    rows = lax.broadcasted_iota(jnp.int32, (_GTM, _GTM), 0)
    cols = lax.broadcasted_iota(jnp.int32, (_GTM, _GTM), 1)
    tril = (cols < rows).astype(jnp.float32)
    ranks = lax.dot_general(tril, ohc, (((1,), (0,)), ((), ())),
                            preferred_element_type=jnp.float32)
    ranks = ranks + cacc_ref[...]
    r1 = jnp.sum(ranks * oh1, axis=1, keepdims=True)
    r2 = jnp.sum(ranks * oh2, axis=1, keepdims=True)

    w1_ref[...] = m1
    w2_ref[...] = m2
    i1_ref[...] = i1[:, 0]
    i2_ref[...] = i2[:, 0]
    r1_ref[...] = r1.astype(jnp.int32)[:, 0]
    r2_ref[...] = r2.astype(jnp.int32)[:, 0]
    cacc_ref[...] += jnp.sum(ohc, axis=0, keepdims=True)
    cnt_ref[...] = cacc_ref[...]


def _gating(x, Wg):
    B, in_dim, n_vars = x.shape
    n_tok = B * n_vars
    vpb = n_vars // _GTM
    grid = (n_tok // _GTM,)
    f32 = jnp.float32
    i32 = jnp.int32
    outs = [jax.ShapeDtypeStruct((n_tok, in_dim), f32),
            jax.ShapeDtypeStruct((n_tok, 1), f32),
            jax.ShapeDtypeStruct((n_tok, 1), f32),
            jax.ShapeDtypeStruct((n_tok,), i32),
            jax.ShapeDtypeStruct((n_tok,), i32),
            jax.ShapeDtypeStruct((n_tok,), i32),
            jax.ShapeDtypeStruct((n_tok,), i32),
            jax.ShapeDtypeStruct((1, _E), f32)]
    tok_spec = pl.BlockSpec((_GTM, 1), lambda i: (i, 0))
    flat_spec = pl.BlockSpec((_GTM,), lambda i: (i,))
    return pl.pallas_call(
        _gating_body,
        grid=grid,
        in_specs=[pl.BlockSpec((1, in_dim, _GTM),
                               lambda i: (i // vpb, 0, i % vpb)),
                  pl.BlockSpec((_E, in_dim), lambda i: (0, 0))],
        out_specs=[pl.BlockSpec((_GTM, in_dim), lambda i: (i, 0)),
                   tok_spec, tok_spec, flat_spec, flat_spec, flat_spec,
                   flat_spec, pl.BlockSpec((1, _E), lambda i: (0, 0))],
        out_shape=outs,
        scratch_shapes=[pltpu.VMEM((1, _E), f32)],
    )(x, Wg)


_TSHIFT = _T.bit_length() - 1


def _aligned_offsets(cv_ref):
    ci = cv_ref[...].astype(jnp.int32)
    av = ((ci + (_T - 1)) >> _TSHIFT) << _TSHIFT
    offs, acc = [], 0
    for e in range(_E):
        offs.append(acc)
        acc = acc + av[e]
    return offs, acc


def _slot_chunk(offs, iv_ref, rv_ref, j):
    ic = iv_ref[pl.ds(j * _SUB, _SUB)]
    s = rv_ref[pl.ds(j * _SUB, _SUB)]
    for e in range(_E):
        s = s + jnp.where(ic == e, offs[e], 0)
    return s


def _sc_scatter_x(xf, i1, i2, r1, r2, c16, n_slots):
    n_tok, d = xf.shape
    per_w = n_tok // _NW
    nsub = per_w // _SUB
    mesh = plsc.VectorSubcoreMesh(core_axis_name="c", subcore_axis_name="s")
    i32 = jnp.int32
    f32 = jnp.float32

    @functools.partial(
        pl.kernel, mesh=mesh,
        out_type=[jax.ShapeDtypeStruct((n_slots, d), f32),
                  jax.ShapeDtypeStruct((_NGID,), i32)],
        scratch_types=[pltpu.VMEM((per_w,), i32), pltpu.VMEM((per_w,), i32),
                       pltpu.VMEM((per_w,), i32), pltpu.VMEM((per_w,), i32),
                       pltpu.VMEM((16,), f32), pltpu.VMEM((_NGID,), i32),
                       pltpu.VMEM((_SUB, d), f32), pltpu.VMEM((_SUB, d), f32),
                       pltpu.SemaphoreType.DMA, pltpu.SemaphoreType.DMA,
                       pltpu.SemaphoreType.DMA],
    )
    def k(x_hbm, i1_hbm, i2_hbm, r1_hbm, r2_hbm, c16_hbm, xs_hbm, gid_hbm,
          i1v, i2v, r1v, r2v, cv, gidv, xv0, xv1,
          lsem, ssem0, ssem1):
        wid = lax.axis_index("s") * 2 + lax.axis_index("c")
        base = wid * per_w
        pltpu.sync_copy(c16_hbm, cv)
        offs, total = _aligned_offsets(cv)
        pltpu.sync_copy(i1_hbm.at[pl.ds(base, per_w)], i1v)
        pltpu.sync_copy(i2_hbm.at[pl.ds(base, per_w)], i2v)
        pltpu.sync_copy(r1_hbm.at[pl.ds(base, per_w)], r1v)
        pltpu.sync_copy(r2_hbm.at[pl.ds(base, per_w)], r2v)

        @pl.when(wid == 0)
        def _gids():
            for c in range(_NGID // 16):
                tstart = (lax.broadcasted_iota(i32, (16,), 0) + c * 16) * _T
                g = jnp.full((16,), -1, i32)
                for e in range(_E):
                    g = g + jnp.where(tstart >= offs[e], 1, 0)
                gidv[pl.ds(c * 16, 16)] = g
            pltpu.sync_copy(gidv, gid_hbm)

        xbufs = (xv0, xv1)
        ssems = (ssem0, ssem1)
        loads = [pltpu.async_copy(x_hbm.at[pl.ds(base, _SUB)], xv0, lsem)]
        scats = []
        for j in range(nsub):
            loads[j].wait()
            s1c = _slot_chunk(offs, i1v, r1v, j)
            s2c = _slot_chunk(offs, i2v, r2v, j)
            xb = xbufs[j % 2]
            scats.append((
                pltpu.async_copy(xb, xs_hbm.at[s1c], ssems[j % 2]),
                pltpu.async_copy(xb, xs_hbm.at[s2c], ssems[j % 2])))
            if j + 1 < nsub:
                if j >= 1:
                    scats[j - 1][0].wait()
                    scats[j - 1][1].wait()
                loads.append(pltpu.async_copy(
                    x_hbm.at[pl.ds(base + (j + 1) * _SUB, _SUB)],
                    xbufs[(j + 1) % 2], lsem))
        for jj in (nsub - 2, nsub - 1):
            if 0 <= jj < nsub:
                scats[jj][0].wait()
                scats[jj][1].wait()

    return k(xf, i1, i2, r1, r2, c16)


def _sc_gather_buf(buf, i1, i2, r1, r2, c16):
    n_slots, d = buf.shape
    n_tok = i1.shape[0]
    per_w = n_tok // _NW
    nsub = 2 * (per_w // _SUB)
    mesh = plsc.VectorSubcoreMesh(core_axis_name="c", subcore_axis_name="s")
    i32 = jnp.int32
    f32 = jnp.float32

    @functools.partial(
        pl.kernel, mesh=mesh,
        out_type=jax.ShapeDtypeStruct((2 * n_tok, d), f32),
        scratch_types=[pltpu.VMEM((per_w,), i32), pltpu.VMEM((per_w,), i32),
                       pltpu.VMEM((per_w,), i32), pltpu.VMEM((per_w,), i32),
                       pltpu.VMEM((16,), f32),
                       pltpu.VMEM((_SUB, d), f32), pltpu.VMEM((_SUB, d), f32),
                       pltpu.SemaphoreType.DMA, pltpu.SemaphoreType.DMA,
                       pltpu.SemaphoreType.DMA, pltpu.SemaphoreType.DMA],
    )
    def k(buf_hbm, i1_hbm, i2_hbm, r1_hbm, r2_hbm, c16_hbm, g12_hbm,
          i1v, i2v, r1v, r2v, cv, gv0, gv1, gsem0, gsem1, stsem0, stsem1):
        wid = lax.axis_index("s") * 2 + lax.axis_index("c")
        base = wid * per_w
        pltpu.sync_copy(c16_hbm, cv)
        offs, _ = _aligned_offsets(cv)
        pltpu.sync_copy(i1_hbm.at[pl.ds(base, per_w)], i1v)
        pltpu.sync_copy(i2_hbm.at[pl.ds(base, per_w)], i2v)
        pltpu.sync_copy(r1_hbm.at[pl.ds(base, per_w)], r1v)
        pltpu.sync_copy(r2_hbm.at[pl.ds(base, per_w)], r2v)

        def slot(j):
            if j % 2 == 0:
                return _slot_chunk(offs, i1v, r1v, j // 2)
            return _slot_chunk(offs, i2v, r2v, j // 2)

        def dst(j):
            half = 0 if j % 2 == 0 else n_tok
            return pl.ds(half + base + (j // 2) * _SUB, _SUB)

        gbufs = (gv0, gv1)
        gsems = (gsem0, gsem1)
        stsems = (stsem0, stsem1)
        gaths = [pltpu.async_copy(buf_hbm.at[slot(0)], gv0, gsem0)]
        stores = []
        for j in range(nsub):
            gaths[j].wait()
            if j + 1 < nsub:
                if j >= 1:
                    stores[j - 1].wait()
                gaths.append(pltpu.async_copy(
                    buf_hbm.at[slot(j + 1)], gbufs[(j + 1) % 2],
                    gsems[(j + 1) % 2]))
            stores.append(pltpu.async_copy(
                gbufs[j % 2], g12_hbm.at[dst(j)], stsems[j % 2]))
        for jj in (nsub - 2, nsub - 1):
            if 0 <= jj < nsub:
                stores[jj].wait()

    return k(buf, i1, i2, r1, r2, c16)


def _gmm_body(gid_ref, xs_ref, we_ref, be_ref, out_ref):
    acc = lax.dot_general(xs_ref[...], we_ref[0],
                          (((1,), (1,)), ((), ())),
                          preferred_element_type=jnp.float32)
    out_ref[...] = acc + be_ref[0]


def _grouped_matmul(gids, Xs, We, be3):
    n_slots, in_dim = Xs.shape
    out_dim = We.shape[1]
    n_tiles = n_slots // _T
    we_map = lambda i, g: (g[i], 0, 0)
    gspec = pltpu.PrefetchScalarGridSpec(
        num_scalar_prefetch=1,
        grid=(n_tiles,),
        in_specs=[
            pl.BlockSpec((_T, in_dim), lambda i, g: (i, 0)),
            pl.BlockSpec((1, out_dim, in_dim), we_map),
            pl.BlockSpec((1, 1, out_dim), we_map),
        ],
        out_specs=pl.BlockSpec((_T, out_dim), lambda i, g: (i, 0)),
    )
    return pl.pallas_call(
        _gmm_body,
        grid_spec=gspec,
        out_shape=jax.ShapeDtypeStruct((n_slots, out_dim), jnp.float32),
    )(gids, Xs, We, be3)


def _combine_body(g1_ref, g2_ref, w1_ref, w2_ref, out_ref):
    y = w1_ref[...] * g1_ref[...] + w2_ref[...] * g2_ref[...]
    out_ref[0] = y.T


def _combine(g12, w1, w2, B, n_vars):
    n2, d = g12.shape
    n_tok = n2 // 2
    tm = _CTM
    vpb = n_vars // tm
    nb = n_tok // tm
    tok_spec = pl.BlockSpec((tm, 1), lambda i: (i, 0))
    return pl.pallas_call(
        _combine_body,
        grid=(nb,),
        in_specs=[pl.BlockSpec((tm, d), lambda i: (i, 0)),
                  pl.BlockSpec((tm, d), lambda i: (i + nb, 0)),
                  tok_spec, tok_spec],
        out_specs=pl.BlockSpec((1, d, tm), lambda i: (i // vpb, 0, i % vpb)),
        out_shape=jax.ShapeDtypeStruct((B, d, n_vars), jnp.float32),
    )(g12, g12, w1, w2)


def kernel(x, Wg, We, be):
    B, in_len, n_vars = x.shape
    out_dim = We.shape[1]
    n_tok = B * n_vars
    n_slots = 2 * n_tok + _E * _T

    xf, w1, w2, i1, i2, r1, r2, cnt = _gating(x, Wg)
    c16 = jnp.pad(cnt.reshape(_E), (0, 16 - _E))
    Xs, gids = _sc_scatter_x(xf, i1, i2, r1, r2, c16, n_slots)
    be3 = be.reshape(_E, 1, out_dim)
    buf = _grouped_matmul(gids, Xs, We, be3)
    g12 = _sc_gather_buf(buf, i1, i2, r1, r2, c16)
    return _combine(g12, w1, w2, B, n_vars)
